# Initial kernel scaffold; baseline (speedup 1.0000x reference)
import jax
import jax.numpy as jnp
from jax.experimental import pallas as pl


def kernel(x, codebooks):
    raise NotImplementedError("write your pallas kernel here")


# speedup vs baseline: 1.6533x; 1.6533x over previous
import jax
import jax.numpy as jnp
from jax import lax
from jax.experimental import pallas as pl

NUM_QUANTIZERS = 8
CODEBOOK_SIZE = 1024
DIM = 64
T_BLK = 512


def _rvq_kernel(x_ref, cb_ref, cbn_ref, out_ref):
    r = x_ref[0]
    acc = jnp.zeros_like(r)
    t_blk = r.shape[1]
    for q in range(NUM_QUANTIZERS):
        cb = cb_ref[q]
        cbn = cbn_ref[q]
        cb_hi = cb.astype(jnp.bfloat16)
        rem1 = cb - cb_hi.astype(jnp.float32)
        cb_mid = rem1.astype(jnp.bfloat16)
        cb_lo = (rem1 - cb_mid.astype(jnp.float32)).astype(jnp.bfloat16)
        rn = jnp.sum(r * r, axis=0)
        scores = lax.dot_general(
            r.astype(jnp.bfloat16), cb_hi, (((0,), (1,)), ((), ())),
            preferred_element_type=jnp.float32,
        )
        dist = (rn[:, None] - 2.0 * scores) + cbn[None, :]
        ind = jnp.argmin(dist, axis=1)
        onehot = (
            lax.broadcasted_iota(jnp.int32, (t_blk, CODEBOOK_SIZE), 1)
            == ind[:, None]
        ).astype(jnp.bfloat16)
        g0, g1, g2 = [
            lax.dot_general(
                part, onehot, (((0,), (1,)), ((), ())),
                preferred_element_type=jnp.float32,
            )
            for part in (cb_hi, cb_mid, cb_lo)
        ]
        qv = (g0 + g1) + g2
        qv_st = r + (qv - r)
        r = r - qv_st
        acc = acc + qv_st
    out_ref[0] = acc


@jax.jit
def kernel(x, codebooks):
    B, D, T = x.shape
    cbn = jnp.sum(codebooks * codebooks, axis=-1)
    grid = (B, T // T_BLK)
    return pl.pallas_call(
        _rvq_kernel,
        grid=grid,
        in_specs=[
            pl.BlockSpec((1, D, T_BLK), lambda b, t: (b, 0, t)),
            pl.BlockSpec(
                (NUM_QUANTIZERS, CODEBOOK_SIZE, DIM), lambda b, t: (0, 0, 0)
            ),
            pl.BlockSpec((NUM_QUANTIZERS, CODEBOOK_SIZE), lambda b, t: (0, 0)),
        ],
        out_specs=pl.BlockSpec((1, D, T_BLK), lambda b, t: (b, 0, t)),
        out_shape=jax.ShapeDtypeStruct((B, D, T), jnp.float32),
    )(x, codebooks, cbn)


# speedup vs baseline: 1.8871x; 1.1414x over previous
import jax
import jax.numpy as jnp
from jax import lax
from jax.experimental import pallas as pl

NUM_QUANTIZERS = 8
CODEBOOK_SIZE = 1024
DIM = 64
T_BLK = 1024


def _rvq_kernel(x_ref, cb_ref, cbn_ref, out_ref):
    r = x_ref[0]
    acc = jnp.zeros_like(r)
    t_blk = r.shape[1]
    for q in range(NUM_QUANTIZERS):
        cb = cb_ref[q]
        cbn = cbn_ref[q]
        cb_hi = cb.astype(jnp.bfloat16)
        rem1 = cb - cb_hi.astype(jnp.float32)
        cb_mid = rem1.astype(jnp.bfloat16)
        cb_lo = (rem1 - cb_mid.astype(jnp.float32)).astype(jnp.bfloat16)
        rn = jnp.sum(r * r, axis=0)
        scores = lax.dot_general(
            r.astype(jnp.bfloat16), cb_hi, (((0,), (1,)), ((), ())),
            preferred_element_type=jnp.float32,
        )
        dist = (rn[:, None] - 2.0 * scores) + cbn[None, :]
        ind = jnp.argmin(dist, axis=1)
        onehot = (
            lax.broadcasted_iota(jnp.int32, (t_blk, CODEBOOK_SIZE), 1)
            == ind[:, None]
        ).astype(jnp.bfloat16)
        g0, g1, g2 = [
            lax.dot_general(
                part, onehot, (((0,), (1,)), ((), ())),
                preferred_element_type=jnp.float32,
            )
            for part in (cb_hi, cb_mid, cb_lo)
        ]
        qv = (g0 + g1) + g2
        qv_st = r + (qv - r)
        r = r - qv_st
        acc = acc + qv_st
    out_ref[0] = acc


@jax.jit
def kernel(x, codebooks):
    B, D, T = x.shape
    cbn = jnp.sum(codebooks * codebooks, axis=-1)
    grid = (B, T // T_BLK)
    return pl.pallas_call(
        _rvq_kernel,
        grid=grid,
        in_specs=[
            pl.BlockSpec((1, D, T_BLK), lambda b, t: (b, 0, t)),
            pl.BlockSpec(
                (NUM_QUANTIZERS, CODEBOOK_SIZE, DIM), lambda b, t: (0, 0, 0)
            ),
            pl.BlockSpec((NUM_QUANTIZERS, CODEBOOK_SIZE), lambda b, t: (0, 0)),
        ],
        out_specs=pl.BlockSpec((1, D, T_BLK), lambda b, t: (b, 0, t)),
        out_shape=jax.ShapeDtypeStruct((B, D, T), jnp.float32),
    )(x, codebooks, cbn)
